# baseline (speedup 1.0000x reference)
import functools

import jax
import jax.numpy as jnp
import numpy as np
from jax import lax
from jax.experimental import pallas as pl
from jax.experimental.pallas import tpu as pltpu
from jax.experimental.pallas import tpu_sc as plsc

_N = 10000
_E = 320000
_G = 256
_D = 128
_H = 32
_NT = 64
_PH = 32
_BNC = float(1.0 / np.sqrt(np.float32(1.0 + 1e-5)))

_NC = 2
_NS = 16
_NW = _NC * _NS
_CH = 125
_NCHUNK = (2 * _E) // _NW // _CH
_K = 8
_NPAD = 20480
_RPT = _NPAD // _NS
_RH = _RPT // 4
_RB = 2000
_NB = _N // _RB

_pcall = pl.pallas_call


def _sc_edge_aggregate(tab, src_r, dst_r):
    mesh = plsc.VectorSubcoreMesh(core_axis_name="c", subcore_axis_name="s")

    @functools.partial(
        pl.kernel,
        out_type=jax.ShapeDtypeStruct((_NC, _NPAD, _H), jnp.float32),
        mesh=mesh,
        scratch_types=[
            pltpu.VMEM((_NCHUNK, _CH), jnp.int32),
            pltpu.VMEM((_NCHUNK, _CH), jnp.int32),
            pltpu.VMEM((_K, _CH, _H), jnp.float32),
            pltpu.VMEM((_RH, _H), jnp.float32),
            pltpu.VMEM_SHARED((_NPAD, _H), jnp.float32),
        ] + [pltpu.SemaphoreType.DMA] * _K,
        compiler_params=pltpu.CompilerParams(use_tc_tiling_on_sc=False),
    )
    def agg(tab_hbm, src_hbm, dst_hbm, out_hbm, srcv, dstv, rows, zbuf, acc,
            *gsem):
        c = lax.axis_index("c")
        s = lax.axis_index("s")
        wid = s * _NC + c

        zeros16 = jnp.zeros((16,), jnp.float32)

        @pl.loop(0, _RH)
        def _(i):
            zbuf[i, pl.ds(0, 16)] = zeros16
            zbuf[i, pl.ds(16, 16)] = zeros16

        for q in range(4):
            pltpu.sync_copy(zbuf, acc.at[pl.ds(s * _RPT + q * _RH, _RH)])
        plsc.subcore_barrier()

        pltpu.sync_copy(src_hbm.at[wid], srcv)
        pltpu.sync_copy(dst_hbm.at[wid], dstv)

        for b in range(_K):
            pltpu.async_copy(tab_hbm.at[srcv.at[b]], rows.at[b], gsem[b])

        @pl.loop(0, _NCHUNK // _K)
        def _(g):
            base = g * _K
            for b in range(_K):
                j = base + b
                pltpu.make_async_copy(
                    tab_hbm.at[pl.ds(0, _CH)], rows.at[b], gsem[b]).wait()
                pltpu.sync_copy(rows.at[b], acc.at[dstv.at[j]], add=True)

                @pl.when(j + _K < _NCHUNK)
                def _():
                    pltpu.async_copy(
                        tab_hbm.at[srcv.at[j + _K]], rows.at[b], gsem[b])

        plsc.subcore_barrier()
        for q in range(4):
            pltpu.sync_copy(acc.at[pl.ds(s * _RPT + q * _RH, _RH)], zbuf)
            pltpu.sync_copy(zbuf, out_hbm.at[c, pl.ds(s * _RPT + q * _RH, _RH)])

    return agg(tab, src_r, dst_r)


def _tc_dense1(xs, Ws, Wrs, brs):
    def body(x_ref, W_ref, Wr_ref, br_ref, hpre_ref, res_ref):
        x = x_ref[0]
        hpre_ref[0] = jnp.dot(x, W_ref[0], preferred_element_type=jnp.float32)
        r = jnp.dot(x, Wr_ref[0], preferred_element_type=jnp.float32) + br_ref[0]
        res_ref[0] = jnp.maximum(r, 0.0)

    return _pcall(
        body,
        grid=(2, _NB),
        in_specs=[
            pl.BlockSpec((1, _RB, _D), lambda p, i: (p, i, 0)),
            pl.BlockSpec((1, _D, _H), lambda p, i: (p, 0, 0)),
            pl.BlockSpec((1, _D, _H), lambda p, i: (p, 0, 0)),
            pl.BlockSpec((1, 1, _H), lambda p, i: (p, 0, 0)),
        ],
        out_specs=[
            pl.BlockSpec((1, _RB, _H), lambda p, i: (p, i, 0)),
            pl.BlockSpec((1, _RB, _H), lambda p, i: (p, i, 0)),
        ],
        out_shape=[jax.ShapeDtypeStruct((2, _N, _H), jnp.float32)] * 2,
    )(xs, Ws, Wrs, brs)


def _tc_dense2(parts, res1, b1s, g1s, be1s, W2s, Wr2s, br2s):
    def body(pr, res_ref, b1, g1, be1, W2, Wr2, br2, hpre2_ref, res2_ref):
        agg = pr[0] + pr[1]
        h1 = (jnp.maximum(agg + b1[0], 0.0) + res_ref[0]) * (g1[0] * _BNC) + be1[0]
        hpre2_ref[0] = jnp.dot(h1, W2[0], preferred_element_type=jnp.float32)
        r = jnp.dot(h1, Wr2[0], preferred_element_type=jnp.float32) + br2[0]
        res2_ref[0] = jnp.maximum(r, 0.0)

    return _pcall(
        body,
        grid=(2, _NB),
        in_specs=[
            pl.BlockSpec((_NC, _RB, _H), lambda p, i: (0, p * _NB + i, 0)),
            pl.BlockSpec((1, _RB, _H), lambda p, i: (p, i, 0)),
            pl.BlockSpec((1, 1, _H), lambda p, i: (p, 0, 0)),
            pl.BlockSpec((1, 1, _H), lambda p, i: (p, 0, 0)),
            pl.BlockSpec((1, 1, _H), lambda p, i: (p, 0, 0)),
            pl.BlockSpec((1, _H, _H), lambda p, i: (p, 0, 0)),
            pl.BlockSpec((1, _H, _H), lambda p, i: (p, 0, 0)),
            pl.BlockSpec((1, 1, _H), lambda p, i: (p, 0, 0)),
        ],
        out_specs=[
            pl.BlockSpec((1, _RB, _H), lambda p, i: (p, i, 0)),
            pl.BlockSpec((1, _RB, _H), lambda p, i: (p, i, 0)),
        ],
        out_shape=[jax.ShapeDtypeStruct((2, _N, _H), jnp.float32)] * 2,
    )(parts, res1, b1s, g1s, be1s, W2s, Wr2s, br2s)


def _tc_readout(parts, res2, b2s, g2s, be2s, Wgs, bgs, gidr):
    def body(pr, res_ref, b2, g2, be2, Wg, bg, gid_ref, hsum_ref, maxT_ref):
        i = pl.program_id(1)
        agg = pr[0] + pr[1]
        h2 = (jnp.maximum(agg + b2[0], 0.0) + res_ref[0]) * (g2[0] * _BNC) + be2[0]
        h2b = h2.astype(jnp.bfloat16).astype(jnp.float32)
        wgb = Wg[0].astype(jnp.bfloat16).astype(jnp.float32)
        logit = jnp.sum(h2b * wgb, axis=1, keepdims=True) + bg[0]
        w = jax.nn.sigmoid(logit)
        wh = w * h2
        gid = gid_ref[0]
        iota_g = lax.broadcasted_iota(jnp.int32, (1, _G), 1)
        ohf = (gid == iota_g).astype(jnp.float32)
        contrib = lax.dot_general(
            ohf, wh, (((0,), (0,)), ((), ())),
            preferred_element_type=jnp.float32,
            precision=lax.Precision.HIGHEST)
        val = h2
        s = 1
        while s < _RB:
            sh_val = jnp.concatenate(
                [jnp.full((s, _H), -jnp.inf, jnp.float32), val[:-s]], axis=0)
            sh_gid = jnp.concatenate(
                [jnp.full((s, 1), -1, jnp.int32), gid[:-s]], axis=0)
            val = jnp.maximum(
                val, jnp.where(sh_gid == gid, sh_val, -jnp.inf))
            s *= 2
        nxt_gid = jnp.concatenate(
            [gid[1:], jnp.full((1, 1), -2, jnp.int32)], axis=0)
        lastf = (gid != nxt_gid).astype(jnp.float32)
        ext = lax.dot_general(
            ohf * lastf,
            jnp.concatenate([val, jnp.ones((_RB, 1), jnp.float32)], axis=1),
            (((0,), (0,)), ((), ())),
            preferred_element_type=jnp.float32,
            precision=lax.Precision.HIGHEST)
        M = jnp.where(ext[:, _H:_H + 1] > 0.0, ext[:, :_H], -jnp.inf)

        @pl.when(i == 0)
        def _():
            hsum_ref[0] = contrib
            maxT_ref[0] = M

        @pl.when(i != 0)
        def _():
            hsum_ref[0] += contrib
            maxT_ref[0] = jnp.maximum(maxT_ref[0], M)

    return _pcall(
        body,
        grid=(2, _NB),
        in_specs=[
            pl.BlockSpec((_NC, _RB, _H), lambda p, i: (0, p * _NB + i, 0)),
            pl.BlockSpec((1, _RB, _H), lambda p, i: (p, i, 0)),
            pl.BlockSpec((1, 1, _H), lambda p, i: (p, 0, 0)),
            pl.BlockSpec((1, 1, _H), lambda p, i: (p, 0, 0)),
            pl.BlockSpec((1, 1, _H), lambda p, i: (p, 0, 0)),
            pl.BlockSpec((1, 1, _H), lambda p, i: (p, 0, 0)),
            pl.BlockSpec((1, 1, 1), lambda p, i: (p, 0, 0)),
            pl.BlockSpec((1, _RB, 1), lambda p, i: (p, i, 0)),
        ],
        out_specs=[
            pl.BlockSpec((1, _G, _H), lambda p, i: (p, 0, 0)),
            pl.BlockSpec((1, _G, _H), lambda p, i: (p, 0, 0)),
        ],
        out_shape=[
            jax.ShapeDtypeStruct((2, _G, _H), jnp.float32),
            jax.ShapeDtypeStruct((2, _G, _H), jnp.float32),
        ],
    )(parts, res2, b2s, g2s, be2s, Wgs, bgs, gidr)


def _tc_head(hsum, maxT, Wm1s, bm1s, gms, bems, Wm2s, bm2s, Wp1, bp1, Wp2, bp2):
    def body(hsum_ref, maxT_ref, Wm1, bm1, gm, bem, Wm2, bm2,
             Wp1_ref, bp1_ref, Wp2_ref, bp2_ref, out_ref):
        fs = []
        for p in range(2):
            hs = hsum_ref[p]
            hm = maxT_ref[p]
            hm = jnp.where(hm > -jnp.inf, hm, 0.0)
            z = jnp.dot(hs, Wm1[p, :_H], preferred_element_type=jnp.float32)
            z = z + jnp.dot(hm, Wm1[p, _H:],
                            preferred_element_type=jnp.float32)
            z = jnp.maximum(z + bm1[p], 0.0)
            z = z * (gm[p] * _BNC) + bem[p]
            fs.append(jnp.dot(z, Wm2[p], preferred_element_type=jnp.float32)
                      + bm2[p])
        f = jnp.concatenate(fs, axis=1)
        zt = jnp.maximum(
            jnp.dot(f, Wp1_ref[...], preferred_element_type=jnp.float32)
            + bp1_ref[...], 0.0)
        ztb = zt.astype(jnp.bfloat16).astype(jnp.float32)
        wpb = Wp2_ref[...].astype(jnp.bfloat16).astype(jnp.float32)
        out_ref[...] = (jnp.sum(ztb * wpb, axis=1, keepdims=True)
                        + bp2_ref[...])

    return _pcall(
        body,
        out_shape=jax.ShapeDtypeStruct((_G, 1), jnp.float32),
    )(hsum, maxT, Wm1s, bm1s, gms, bems, Wm2s, bm2s, Wp1, bp1, Wp2, bp2)


def kernel(x0, x1, edge_index0, edge_index1, graph_id0, graph_id1,
           params1, params2, head):
    st = lambda k: jnp.stack([params1[k], params2[k]])
    vt = lambda k: jnp.stack([params1[k], params2[k]])[:, None, :]

    xs = jnp.stack([x0, x1])
    src_r = jnp.concatenate(
        [edge_index0[0], edge_index1[0] + _N]).reshape(_NW, _NCHUNK, _CH)
    dst_r = jnp.concatenate(
        [edge_index0[1], edge_index1[1] + _N]).reshape(_NW, _NCHUNK, _CH)
    gidr = jnp.stack([graph_id0, graph_id1])[:, :, None]

    hpre1, res1 = _tc_dense1(xs, st('W1'), st('Wr1'), vt('br1'))
    parts1 = _sc_edge_aggregate(hpre1.reshape(2 * _N, _H), src_r, dst_r)
    hpre2, res2 = _tc_dense2(parts1, res1, vt('b1'), vt('g1'), vt('be1'),
                             st('W2'), st('Wr2'), vt('br2'))
    parts2 = _sc_edge_aggregate(hpre2.reshape(2 * _N, _H), src_r, dst_r)
    hsum, maxT = _tc_readout(parts2, res2, vt('b2'), vt('g2'), vt('be2'),
                             st('Wg')[:, :, 0][:, None, :],
                             st('bg')[:, :, None], gidr)
    return _tc_head(hsum, maxT, st('Wm1'), vt('bm1'), vt('gm'), vt('bem'),
                    st('Wm2'), vt('bm2'),
                    head['Wp1'], head['bp1'][None, :],
                    head['Wp2'].reshape(1, _NT), head['bp2'][None, :])


# speedup vs baseline: 1.1906x; 1.1906x over previous
import functools

import jax
import jax.numpy as jnp
import numpy as np
from jax import lax
from jax.experimental import pallas as pl
from jax.experimental.pallas import tpu as pltpu
from jax.experimental.pallas import tpu_sc as plsc

_N = 10000
_E = 320000
_G = 256
_D = 128
_H = 32
_NT = 64
_PH = 32
_BNC = float(1.0 / np.sqrt(np.float32(1.0 + 1e-5)))

_NC = 2
_NS = 16
_CH = 125
_NCHUNK = _E // _NS // _CH
_K = 8
_NPAD = 10240
_RPT = _NPAD // _NS
_RH = _RPT // 4
_RB = 2000
_NB = _N // _RB

_pcall = pl.pallas_call


def _sc_edge_aggregate(tab, e0, e1):
    mesh = plsc.VectorSubcoreMesh(core_axis_name="c", subcore_axis_name="s")

    @functools.partial(
        pl.kernel,
        out_type=jax.ShapeDtypeStruct((_NC, _NPAD, _H), jnp.float32),
        mesh=mesh,
        scratch_types=[
            pltpu.VMEM((_NCHUNK, _CH), jnp.int32),
            pltpu.VMEM((_NCHUNK, _CH), jnp.int32),
            pltpu.VMEM((_K, _CH, _H), jnp.float32),
            pltpu.VMEM((_RH, _H), jnp.float32),
            pltpu.VMEM_SHARED((_NPAD, _H), jnp.float32),
        ] + [pltpu.SemaphoreType.DMA] * _K,
        compiler_params=pltpu.CompilerParams(use_tc_tiling_on_sc=False),
    )
    def agg(tab_hbm, e0_hbm, e1_hbm, out_hbm, srcv, dstv, rows, zbuf, acc,
            *gsem):
        c = lax.axis_index("c")
        s = lax.axis_index("s")

        zeros16 = jnp.zeros((16,), jnp.float32)

        @pl.loop(0, _RH)
        def _(i):
            zbuf[i, pl.ds(0, 16)] = zeros16
            zbuf[i, pl.ds(16, 16)] = zeros16

        for q in range(4):
            pltpu.sync_copy(zbuf, acc.at[pl.ds(s * _RPT + q * _RH, _RH)])
        plsc.subcore_barrier()

        @pl.when(c == 0)
        def _():
            pltpu.sync_copy(e0_hbm.at[0, s], srcv)
            pltpu.sync_copy(e0_hbm.at[1, s], dstv)

        @pl.when(c == 1)
        def _():
            pltpu.sync_copy(e1_hbm.at[0, s], srcv)
            pltpu.sync_copy(e1_hbm.at[1, s], dstv)

        for b in range(_K):
            pltpu.async_copy(tab_hbm.at[srcv.at[b]], rows.at[b], gsem[b])

        @pl.loop(0, _NCHUNK // _K)
        def _(g):
            base = g * _K
            for b in range(_K):
                j = base + b
                pltpu.make_async_copy(
                    tab_hbm.at[pl.ds(0, _CH)], rows.at[b], gsem[b]).wait()
                pltpu.sync_copy(rows.at[b], acc.at[dstv.at[j]], add=True)

                @pl.when(j + _K < _NCHUNK)
                def _():
                    pltpu.async_copy(
                        tab_hbm.at[srcv.at[j + _K]], rows.at[b], gsem[b])

        plsc.subcore_barrier()
        for q in range(4):
            pltpu.sync_copy(acc.at[pl.ds(s * _RPT + q * _RH, _RH)], zbuf)
            pltpu.sync_copy(zbuf, out_hbm.at[c, pl.ds(s * _RPT + q * _RH, _RH)])

    return agg(tab, e0, e1)


def _tc_dense1(x0, x1, Ws, Wrs, brs):
    def body(x0_ref, x1_ref, W_ref, Wr_ref, br_ref, hpre_ref, res_ref):
        p = pl.program_id(0)
        x = jnp.where(p == 0, x0_ref[...], x1_ref[...])
        hpre_ref[0] = jnp.dot(x, W_ref[0], preferred_element_type=jnp.float32)
        r = jnp.dot(x, Wr_ref[0], preferred_element_type=jnp.float32) + br_ref[0]
        res_ref[0] = jnp.maximum(r, 0.0)

    return _pcall(
        body,
        grid=(2, _NB),
        in_specs=[
            pl.BlockSpec((_RB, _D), lambda p, i: ((1 - p) * i, 0)),
            pl.BlockSpec((_RB, _D), lambda p, i: (p * i, 0)),
            pl.BlockSpec((1, _D, _H), lambda p, i: (p, 0, 0)),
            pl.BlockSpec((1, _D, _H), lambda p, i: (p, 0, 0)),
            pl.BlockSpec((1, 1, _H), lambda p, i: (p, 0, 0)),
        ],
        out_specs=[
            pl.BlockSpec((1, _RB, _H), lambda p, i: (p, i, 0)),
            pl.BlockSpec((1, _RB, _H), lambda p, i: (p, i, 0)),
        ],
        out_shape=[jax.ShapeDtypeStruct((2, _N, _H), jnp.float32)] * 2,
    )(x0, x1, Ws, Wrs, brs)


def _tc_dense2(parts, res1, b1s, g1s, be1s, W2s, Wr2s, br2s):
    def body(pr, res_ref, b1, g1, be1, W2, Wr2, br2, hpre2_ref, res2_ref):
        agg = pr[0]
        h1 = (jnp.maximum(agg + b1[0], 0.0) + res_ref[0]) * (g1[0] * _BNC) + be1[0]
        hpre2_ref[0] = jnp.dot(h1, W2[0], preferred_element_type=jnp.float32)
        r = jnp.dot(h1, Wr2[0], preferred_element_type=jnp.float32) + br2[0]
        res2_ref[0] = jnp.maximum(r, 0.0)

    return _pcall(
        body,
        grid=(2, _NB),
        in_specs=[
            pl.BlockSpec((1, _RB, _H), lambda p, i: (p, i, 0)),
            pl.BlockSpec((1, _RB, _H), lambda p, i: (p, i, 0)),
            pl.BlockSpec((1, 1, _H), lambda p, i: (p, 0, 0)),
            pl.BlockSpec((1, 1, _H), lambda p, i: (p, 0, 0)),
            pl.BlockSpec((1, 1, _H), lambda p, i: (p, 0, 0)),
            pl.BlockSpec((1, _H, _H), lambda p, i: (p, 0, 0)),
            pl.BlockSpec((1, _H, _H), lambda p, i: (p, 0, 0)),
            pl.BlockSpec((1, 1, _H), lambda p, i: (p, 0, 0)),
        ],
        out_specs=[
            pl.BlockSpec((1, _RB, _H), lambda p, i: (p, i, 0)),
            pl.BlockSpec((1, _RB, _H), lambda p, i: (p, i, 0)),
        ],
        out_shape=[jax.ShapeDtypeStruct((2, _N, _H), jnp.float32)] * 2,
    )(parts, res1, b1s, g1s, be1s, W2s, Wr2s, br2s)


def _tc_readout(parts, res2, b2s, g2s, be2s, Wgs, bgs, gidr):
    def body(pr, res_ref, b2, g2, be2, Wg, bg, gid_ref, hsum_ref, maxT_ref):
        i = pl.program_id(1)
        agg = pr[0]
        h2 = (jnp.maximum(agg + b2[0], 0.0) + res_ref[0]) * (g2[0] * _BNC) + be2[0]
        h2b = h2.astype(jnp.bfloat16).astype(jnp.float32)
        wgb = Wg[0].astype(jnp.bfloat16).astype(jnp.float32)
        logit = jnp.sum(h2b * wgb, axis=1, keepdims=True) + bg[0]
        w = jax.nn.sigmoid(logit)
        wh = w * h2
        gid = gid_ref[0]
        iota_g = lax.broadcasted_iota(jnp.int32, (1, _G), 1)
        ohf = (gid == iota_g).astype(jnp.float32)
        contrib = lax.dot_general(
            ohf, wh, (((0,), (0,)), ((), ())),
            preferred_element_type=jnp.float32,
            precision=lax.Precision.HIGHEST)
        val = h2
        s = 1
        while s < _RB:
            sh_val = jnp.concatenate(
                [jnp.full((s, _H), -jnp.inf, jnp.float32), val[:-s]], axis=0)
            sh_gid = jnp.concatenate(
                [jnp.full((s, 1), -1, jnp.int32), gid[:-s]], axis=0)
            val = jnp.maximum(
                val, jnp.where(sh_gid == gid, sh_val, -jnp.inf))
            s *= 2
        nxt_gid = jnp.concatenate(
            [gid[1:], jnp.full((1, 1), -2, jnp.int32)], axis=0)
        lastf = (gid != nxt_gid).astype(jnp.float32)
        ext = lax.dot_general(
            ohf * lastf,
            jnp.concatenate([val, jnp.ones((_RB, 1), jnp.float32)], axis=1),
            (((0,), (0,)), ((), ())),
            preferred_element_type=jnp.float32,
            precision=lax.Precision.HIGHEST)
        M = jnp.where(ext[:, _H:_H + 1] > 0.0, ext[:, :_H], -jnp.inf)

        @pl.when(i == 0)
        def _():
            hsum_ref[0] = contrib
            maxT_ref[0] = M

        @pl.when(i != 0)
        def _():
            hsum_ref[0] += contrib
            maxT_ref[0] = jnp.maximum(maxT_ref[0], M)

    return _pcall(
        body,
        grid=(2, _NB),
        in_specs=[
            pl.BlockSpec((1, _RB, _H), lambda p, i: (p, i, 0)),
            pl.BlockSpec((1, _RB, _H), lambda p, i: (p, i, 0)),
            pl.BlockSpec((1, 1, _H), lambda p, i: (p, 0, 0)),
            pl.BlockSpec((1, 1, _H), lambda p, i: (p, 0, 0)),
            pl.BlockSpec((1, 1, _H), lambda p, i: (p, 0, 0)),
            pl.BlockSpec((1, 1, _H), lambda p, i: (p, 0, 0)),
            pl.BlockSpec((1, 1, 1), lambda p, i: (p, 0, 0)),
            pl.BlockSpec((1, _RB, 1), lambda p, i: (p, i, 0)),
        ],
        out_specs=[
            pl.BlockSpec((1, _G, _H), lambda p, i: (p, 0, 0)),
            pl.BlockSpec((1, _G, _H), lambda p, i: (p, 0, 0)),
        ],
        out_shape=[
            jax.ShapeDtypeStruct((2, _G, _H), jnp.float32),
            jax.ShapeDtypeStruct((2, _G, _H), jnp.float32),
        ],
    )(parts, res2, b2s, g2s, be2s, Wgs, bgs, gidr)


def _tc_head(hsum, maxT, Wm1s, bm1s, gms, bems, Wm2s, bm2s, Wp1, bp1, Wp2, bp2):
    def body(hsum_ref, maxT_ref, Wm1, bm1, gm, bem, Wm2, bm2,
             Wp1_ref, bp1_ref, Wp2_ref, bp2_ref, out_ref):
        fs = []
        for p in range(2):
            hs = hsum_ref[p]
            hm = maxT_ref[p]
            hm = jnp.where(hm > -jnp.inf, hm, 0.0)
            z = jnp.dot(hs, Wm1[p, :_H], preferred_element_type=jnp.float32)
            z = z + jnp.dot(hm, Wm1[p, _H:],
                            preferred_element_type=jnp.float32)
            z = jnp.maximum(z + bm1[p], 0.0)
            z = z * (gm[p] * _BNC) + bem[p]
            fs.append(jnp.dot(z, Wm2[p], preferred_element_type=jnp.float32)
                      + bm2[p])
        f = jnp.concatenate(fs, axis=1)
        zt = jnp.maximum(
            jnp.dot(f, Wp1_ref[...], preferred_element_type=jnp.float32)
            + bp1_ref[...], 0.0)
        ztb = zt.astype(jnp.bfloat16).astype(jnp.float32)
        wpb = Wp2_ref[...].astype(jnp.bfloat16).astype(jnp.float32)
        out_ref[...] = (jnp.sum(ztb * wpb, axis=1, keepdims=True)
                        + bp2_ref[...])

    return _pcall(
        body,
        out_shape=jax.ShapeDtypeStruct((_G, 1), jnp.float32),
    )(hsum, maxT, Wm1s, bm1s, gms, bems, Wm2s, bm2s, Wp1, bp1, Wp2, bp2)


def kernel(x0, x1, edge_index0, edge_index1, graph_id0, graph_id1,
           params1, params2, head):
    st = lambda k: jnp.stack([params1[k], params2[k]])
    vt = lambda k: jnp.stack([params1[k], params2[k]])[:, None, :]

    e0 = edge_index0.reshape(2, _NS, _NCHUNK, _CH)
    e1 = (edge_index1 + jnp.array([[_N], [0]], jnp.int32)
          ).reshape(2, _NS, _NCHUNK, _CH)
    gidr = jnp.stack([graph_id0, graph_id1])[:, :, None]

    hpre1, res1 = _tc_dense1(x0, x1, st('W1'), st('Wr1'), vt('br1'))
    parts1 = _sc_edge_aggregate(hpre1.reshape(2 * _N, _H), e0, e1)
    hpre2, res2 = _tc_dense2(parts1, res1, vt('b1'), vt('g1'), vt('be1'),
                             st('W2'), st('Wr2'), vt('br2'))
    parts2 = _sc_edge_aggregate(hpre2.reshape(2 * _N, _H), e0, e1)
    hsum, maxT = _tc_readout(parts2, res2, vt('b2'), vt('g2'), vt('be2'),
                             st('Wg')[:, :, 0][:, None, :],
                             st('bg')[:, :, None], gidr)
    return _tc_head(hsum, maxT, st('Wm1'), vt('bm1'), vt('gm'), vt('bem'),
                    st('Wm2'), vt('bm2'),
                    head['Wp1'], head['bp1'][None, :],
                    head['Wp2'].reshape(1, _NT), head['bp2'][None, :])
